# baseline (speedup 1.0000x reference)
import jax
import jax.numpy as jnp
import numpy as np
from jax import lax
from jax.experimental import pallas as pl
from jax.experimental.pallas import tpu as pltpu
from jax.experimental.pallas import tpu_sc as plsc

EPS_ = 1e-07
H_ = 2048
N_ = 16384
Q_ = 0.841

AQ_ = np.float32(Q_) * np.float32(N_ - 1)
J_UPP = int(np.floor(AQ_))
FRAC_ = float(AQ_ - np.float32(J_UPP))
RANKS = (N_ // 2 - 1, N_ // 2, J_UPP, J_UPP + 1)
NR = 4

NTILES = 32
CPT = H_ // NTILES
T_ = 256
NCHUNK = N_ // T_
L = 16
NG = CPT // L
HWORDS = CPT * NR * 256

SIGN_ = np.uint32(0x80000000)
LOW31_ = np.uint32(0x7FFFFFFF)
TOPMASK = (None, np.uint32(0xFF000000), np.uint32(0xFFFF0000),
           np.uint32(0xFFFFFF00))


def _ukey(v):
    b = lax.bitcast_convert_type(v, jnp.int32)
    m = lax.shift_right_arithmetic(b, 31)
    k = b ^ lax.shift_right_logical(m, 1)
    return lax.bitcast_convert_type(k, jnp.uint32) ^ SIGN_


def _ukey_to_f32(u):
    k = u ^ SIGN_
    m = k >> 31
    return lax.bitcast_convert_type(k ^ (m * LOW31_), jnp.float32)


def _sc_body(x_hbm, med_hbm, upp_hbm, sc_hbm, om_hbm, os_hbm,
             buf0_v, buf1_v, hist_v, pref_v, rank_v, med_v, upp_v,
             scal_v, outm_v, outs_v, sem0, sem1):
    cid = lax.axis_index("c")
    sid = lax.axis_index("s")
    wid = sid * 2 + cid
    c0 = wid * CPT

    bufs = (buf0_v, buf1_v)
    sems = (sem0, sem1)

    iota = lax.iota(jnp.int32, L)
    zeros_i = jnp.zeros((L,), jnp.int32)
    zeros_u = jnp.zeros((L,), jnp.uint32)
    ones_i = jnp.ones((L,), jnp.int32)

    def dma(ck, b):
        return pltpu.make_async_copy(
            x_hbm.at[pl.ds(ck * T_, T_), pl.ds(c0, CPT)], bufs[b], sems[b])

    dma(0, 0).start()

    pltpu.sync_copy(med_hbm.at[pl.ds(c0, CPT)], med_v)
    pltpu.sync_copy(upp_hbm.at[pl.ds(c0, CPT)], upp_v)
    pltpu.sync_copy(sc_hbm, scal_v)

    def zero_hist():
        @plsc.parallel_loop(0, HWORDS, step=L, unroll=8)
        def _(i):
            hist_v[pl.ds(i, L)] = zeros_i

    zero_hist()

    for r in range(NR):
        for g in range(NG):
            sl = pl.ds(r * CPT + g * L, L)
            rank_v[sl] = jnp.full((L,), RANKS[r], jnp.int32)
            pref_v[sl] = zeros_u

    for p in range(4):
        shift = 24 - 8 * p
        if p > 0:
            zero_hist()

        def process(buf, p=p, shift=shift):
            for g in range(NG):
                chbase = (iota + g * L) * (NR * 256)
                if p > 0:
                    prefs = [pref_v[pl.ds(r * CPT + g * L, L)]
                             for r in range(NR)]
                    bases = [chbase + r * 256 for r in range(NR)]

                @plsc.parallel_loop(0, T_, unroll=8 if p == 0 else 4)
                def row_body(i, g=g, chbase=chbase):
                    v = buf[i, pl.ds(g * L, L)]
                    u = _ukey(v)
                    dig = ((u >> shift) & np.uint32(0xFF)).astype(jnp.int32)
                    if p == 0:
                        plsc.addupdate_scatter(hist_v, [chbase + dig],
                                               ones_i)
                    else:
                        hi = u & TOPMASK[p]
                        for r in range(NR):
                            plsc.addupdate_scatter(
                                hist_v, [bases[r] + dig], ones_i,
                                mask=hi == prefs[r])

        def pair_body(ip, _, process=process, last=(p == 3)):
            for b in (0, 1):
                ck = 2 * ip + b
                if b == 0:
                    dma(ck + 1, 1).start()
                else:
                    @pl.when(ip < NCHUNK // 2 - 1)
                    def _():
                        dma(ck + 1, 0).start()
                dma(ck, b).wait()
                process(bufs[b])
            return 0

        lax.fori_loop(0, NCHUNK // 2, pair_body, 0)
        if p < 3:
            dma(0, 0).start()

        for g in range(NG):
            chbase = (iota + g * L) * (NR * 256)
            for r in range(NR):
                rsl = 0 if p == 0 else r
                sl = pl.ds(r * CPT + g * L, L)
                rank = rank_v[sl]
                base = chbase + rsl * 256

                def wbody(d, carry, base=base, rank=rank):
                    cdf, dcnt, skip = carry
                    h = plsc.load_gather(hist_v, [base + d])
                    cdf2 = cdf + h
                    le = cdf2 <= rank
                    dcnt2 = dcnt + le.astype(jnp.int32)
                    skip2 = jnp.where(le, cdf2, skip)
                    return cdf2, dcnt2, skip2

                _, dcnt, skip = lax.fori_loop(
                    0, 256, wbody, (zeros_i, zeros_i, zeros_i), unroll=4)
                pref_v[sl] = pref_v[sl] | (dcnt.astype(jnp.uint32) << shift)
                rank_v[sl] = rank - skip

    bpow = scal_v[pl.ds(0, L)]
    dive = scal_v[pl.ds(L, L)]
    trig = scal_v[pl.ds(2 * L, L)]

    for g in range(NG):
        vals = []
        for r in range(NR):
            vals.append(_ukey_to_f32(pref_v[pl.ds(r * CPT + g * L, L)]))
        new_med = 0.5 * (vals[0] + vals[1])
        new_upp = vals[2] * (1.0 - FRAC_) + vals[3] * FRAC_
        med = med_v[pl.ds(g * L, L)]
        upp = upp_v[pl.ds(g * L, L)]
        med_u = bpow * med + (1.0 - bpow) * new_med
        upp_u = bpow * upp + (1.0 - bpow) * new_upp
        med_f = trig * med + (1.0 - trig) * med_u
        upp_f = trig * upp + (1.0 - trig) * upp_u
        adj_med = med_f / dive
        adj_upp = upp_f / dive
        outm_v[pl.ds(g * L, L)] = adj_med
        outs_v[pl.ds(g * L, L)] = adj_upp - adj_med + EPS_

    pltpu.sync_copy(outm_v, om_hbm.at[pl.ds(c0, CPT)])
    pltpu.sync_copy(outs_v, os_hbm.at[pl.ds(c0, CPT)])


@jax.jit
def _run(xr, med, upp, scalars):
    mesh = plsc.VectorSubcoreMesh(core_axis_name="c", subcore_axis_name="s")
    f = pl.kernel(
        _sc_body,
        mesh=mesh,
        compiler_params=pltpu.CompilerParams(use_tc_tiling_on_sc=False,
                                             needs_layout_passes=False),
        out_type=[
            jax.ShapeDtypeStruct((H_,), jnp.float32),
            jax.ShapeDtypeStruct((H_,), jnp.float32),
        ],
        scratch_types=[
            pltpu.VMEM((T_, CPT), jnp.float32),
            pltpu.VMEM((T_, CPT), jnp.float32),
            pltpu.VMEM((HWORDS,), jnp.int32),
            pltpu.VMEM((NR * CPT,), jnp.uint32),
            pltpu.VMEM((NR * CPT,), jnp.int32),
            pltpu.VMEM((CPT,), jnp.float32),
            pltpu.VMEM((CPT,), jnp.float32),
            pltpu.VMEM((3 * L,), jnp.float32),
            pltpu.VMEM((CPT,), jnp.float32),
            pltpu.VMEM((CPT,), jnp.float32),
            pltpu.SemaphoreType.DMA,
            pltpu.SemaphoreType.DMA,
        ],
    )
    return f(xr, med, upp, scalars)


def kernel(x, med, upp, steps, beta):
    xr = x[:4].reshape(N_, H_).astype(jnp.float32)
    delta = 1.0
    bpow = beta ** delta
    trig = (steps > 1.0).astype(jnp.float32)
    steps_f = jnp.where(steps > 1.0, steps, steps + delta)
    dive = 1.0 - beta ** steps_f + EPS_
    scalars = jnp.concatenate([
        jnp.broadcast_to(bpow, (L,)),
        jnp.broadcast_to(dive, (L,)),
        jnp.broadcast_to(trig, (L,)),
    ]).astype(jnp.float32)
    out = _run(xr, med, upp, scalars)
    return (out[0], out[1])


# speedup vs baseline: 1.7654x; 1.7654x over previous
import jax
import jax.numpy as jnp
import numpy as np
from jax import lax
from jax.experimental import pallas as pl
from jax.experimental.pallas import tpu as pltpu
from jax.experimental.pallas import tpu_sc as plsc

EPS_ = 1e-07
H_ = 2048
N_ = 16384
Q_ = 0.841

AQ_ = np.float32(Q_) * np.float32(N_ - 1)
J_UPP = int(np.floor(AQ_))
FRAC_ = float(AQ_ - np.float32(J_UPP))
RANKS = (N_ // 2 - 1, N_ // 2, J_UPP, J_UPP + 1)
NR = 4

H_SC = 1024
H_TC = H_ - H_SC

NTILES = 32
CPT = H_SC // NTILES
T_ = 256
NCHUNK = N_ // T_
L = 16
NG = CPT // L
HWORDS = CPT * NR * 256

BC_ = 256

SIGN_ = np.uint32(0x80000000)
MIN32 = np.int32(-2147483648)
MAX32 = np.int32(2147483647)
TOPMASK = (None, np.uint32(0xFF000000), np.uint32(0xFFFF0000),
           np.uint32(0xFFFFFF00))


def _ema_epilogue(new_med, new_upp, med, upp, bpow, dive, trig):
    med_u = bpow * med + (1.0 - bpow) * new_med
    upp_u = bpow * upp + (1.0 - bpow) * new_upp
    med_f = trig * med + (1.0 - trig) * med_u
    upp_f = trig * upp + (1.0 - trig) * upp_u
    adj_med = med_f / dive
    adj_upp = upp_f / dive
    return adj_med, adj_upp - adj_med + EPS_


def _ukey(v):
    b = lax.bitcast_convert_type(v, jnp.int32)
    m = lax.shift_right_arithmetic(b, 31)
    k = b ^ lax.shift_right_logical(m, 1)
    return lax.bitcast_convert_type(k, jnp.uint32) ^ SIGN_


def _ukey_to_f32(u):
    k = lax.bitcast_convert_type(u ^ SIGN_, jnp.int32)
    m = lax.shift_right_arithmetic(k, 31)
    return lax.bitcast_convert_type(
        k ^ lax.shift_right_logical(m, 1), jnp.float32)


def _sc_body(x_hbm, med_hbm, upp_hbm, sc_hbm, om_hbm, os_hbm,
             buf0_v, buf1_v, hist_v, pref_v, rank_v, med_v, upp_v,
             scal_v, outm_v, outs_v, sem0, sem1):
    cid = lax.axis_index("c")
    sid = lax.axis_index("s")
    wid = sid * 2 + cid
    c0 = wid * CPT

    bufs = (buf0_v, buf1_v)
    sems = (sem0, sem1)

    iota = lax.iota(jnp.int32, L)
    zeros_i = jnp.zeros((L,), jnp.int32)
    zeros_u = jnp.zeros((L,), jnp.uint32)
    ones_i = jnp.ones((L,), jnp.int32)

    def dma(ck, b):
        return pltpu.make_async_copy(
            x_hbm.at[pl.ds(ck * T_, T_), pl.ds(c0, CPT)], bufs[b], sems[b])

    dma(0, 0).start()

    pltpu.sync_copy(med_hbm.at[pl.ds(c0, CPT)], med_v)
    pltpu.sync_copy(upp_hbm.at[pl.ds(c0, CPT)], upp_v)
    pltpu.sync_copy(sc_hbm, scal_v)

    def zero_hist():
        @plsc.parallel_loop(0, HWORDS, step=L, unroll=8)
        def _(i):
            hist_v[pl.ds(i, L)] = zeros_i

    zero_hist()

    for r in range(NR):
        for g in range(NG):
            sl = pl.ds(r * CPT + g * L, L)
            rank_v[sl] = jnp.full((L,), RANKS[r], jnp.int32)
            pref_v[sl] = zeros_u

    for p in range(4):
        shift = 24 - 8 * p
        if p > 0:
            zero_hist()

        def process(buf, p=p, shift=shift):
            for g in range(NG):
                chbase = (iota + g * L) * (NR * 256)
                if p > 0:
                    prefs = [pref_v[pl.ds(r * CPT + g * L, L)]
                             for r in range(NR)]
                    bases = [chbase + r * 256 for r in range(NR)]

                @plsc.parallel_loop(0, T_, unroll=8 if p == 0 else 4)
                def row_body(i, g=g, chbase=chbase):
                    v = buf[i, pl.ds(g * L, L)]
                    u = _ukey(v)
                    dig = ((u >> shift) & np.uint32(0xFF)).astype(jnp.int32)
                    if p == 0:
                        plsc.addupdate_scatter(hist_v, [chbase + dig],
                                               ones_i)
                    else:
                        hi = u & TOPMASK[p]
                        for r in range(NR):
                            plsc.addupdate_scatter(
                                hist_v, [bases[r] + dig], ones_i,
                                mask=hi == prefs[r])

        def pair_body(ip, _, process=process):
            for b in (0, 1):
                ck = 2 * ip + b
                if b == 0:
                    dma(ck + 1, 1).start()
                else:
                    @pl.when(ip < NCHUNK // 2 - 1)
                    def _():
                        dma(ck + 1, 0).start()
                dma(ck, b).wait()
                process(bufs[b])
            return 0

        lax.fori_loop(0, NCHUNK // 2, pair_body, 0)
        if p < 3:
            dma(0, 0).start()

        for g in range(NG):
            chbase = (iota + g * L) * (NR * 256)
            for r in range(NR):
                rsl = 0 if p == 0 else r
                sl = pl.ds(r * CPT + g * L, L)
                rank = rank_v[sl]
                base = chbase + rsl * 256

                def wbody(d, carry, base=base, rank=rank):
                    cdf, dcnt, skip = carry
                    h = plsc.load_gather(hist_v, [base + d])
                    cdf2 = cdf + h
                    le = cdf2 <= rank
                    dcnt2 = dcnt + le.astype(jnp.int32)
                    skip2 = jnp.where(le, cdf2, skip)
                    return cdf2, dcnt2, skip2

                _, dcnt, skip = lax.fori_loop(
                    0, 256, wbody, (zeros_i, zeros_i, zeros_i), unroll=4)
                pref_v[sl] = pref_v[sl] | (dcnt.astype(jnp.uint32) << shift)
                rank_v[sl] = rank - skip

    bpow = scal_v[pl.ds(0, L)]
    dive = scal_v[pl.ds(L, L)]
    trig = scal_v[pl.ds(2 * L, L)]

    for g in range(NG):
        vals = []
        for r in range(NR):
            vals.append(_ukey_to_f32(pref_v[pl.ds(r * CPT + g * L, L)]))
        new_med = 0.5 * (vals[0] + vals[1])
        new_upp = vals[2] * (1.0 - FRAC_) + vals[3] * FRAC_
        med = med_v[pl.ds(g * L, L)]
        upp = upp_v[pl.ds(g * L, L)]
        am, as_ = _ema_epilogue(new_med, new_upp, med, upp,
                                bpow, dive, trig)
        outm_v[pl.ds(g * L, L)] = am
        outs_v[pl.ds(g * L, L)] = as_

    pltpu.sync_copy(outm_v, om_hbm.at[pl.ds(c0, CPT)])
    pltpu.sync_copy(outs_v, os_hbm.at[pl.ds(c0, CPT)])


def _sc_run(xs, med, upp, scalars):
    mesh = plsc.VectorSubcoreMesh(core_axis_name="c", subcore_axis_name="s")
    f = pl.kernel(
        _sc_body,
        mesh=mesh,
        compiler_params=pltpu.CompilerParams(use_tc_tiling_on_sc=False,
                                             needs_layout_passes=False),
        out_type=[
            jax.ShapeDtypeStruct((H_SC,), jnp.float32),
            jax.ShapeDtypeStruct((H_SC,), jnp.float32),
        ],
        scratch_types=[
            pltpu.VMEM((T_, CPT), jnp.float32),
            pltpu.VMEM((T_, CPT), jnp.float32),
            pltpu.VMEM((HWORDS,), jnp.int32),
            pltpu.VMEM((NR * CPT,), jnp.uint32),
            pltpu.VMEM((NR * CPT,), jnp.int32),
            pltpu.VMEM((CPT,), jnp.float32),
            pltpu.VMEM((CPT,), jnp.float32),
            pltpu.VMEM((3 * L,), jnp.float32),
            pltpu.VMEM((CPT,), jnp.float32),
            pltpu.VMEM((CPT,), jnp.float32),
            pltpu.SemaphoreType.DMA,
            pltpu.SemaphoreType.DMA,
        ],
    )
    return f(xs, med, upp, scalars)


def _to_key(b):
    m = lax.shift_right_arithmetic(b, 31)
    return lax.bitwise_xor(b, lax.shift_right_logical(m, 1))


def _select_rank(keys, j):
    ukey_prefix = jnp.zeros((1, BC_), jnp.int32)

    def body(i, p):
        bit = lax.shift_left(jnp.int32(1), (31 - i).astype(jnp.int32))
        cand_u = lax.bitwise_or(p, bit)
        cand_s = cand_u ^ MIN32
        cnt = jnp.sum((keys < cand_s).astype(jnp.int32), axis=0,
                      keepdims=True)
        return jnp.where(cnt <= j, cand_u, p)

    p = lax.fori_loop(0, 32, body, ukey_prefix)
    return p ^ MIN32


def _pair(keys, k_lo, j):
    gt = keys > k_lo
    cnt_gt = jnp.sum(gt.astype(jnp.int32), axis=0, keepdims=True)
    succ = jnp.min(jnp.where(gt, keys, MAX32), axis=0, keepdims=True)
    count_le = N_ - cnt_gt
    k_hi = jnp.where(count_le >= j + 2, k_lo, succ)
    return k_lo, k_hi


def _key_to_f32(k):
    m = lax.shift_right_arithmetic(k, 31)
    b = lax.bitwise_xor(k, lax.shift_right_logical(m, 1))
    return lax.bitcast_convert_type(b, jnp.float32)


def _tc_body(x_ref, med_ref, upp_ref, sc_ref, om_ref, os_ref, keys_ref):
    xb = x_ref[...]
    keys_ref[...] = _to_key(lax.bitcast_convert_type(xb, jnp.int32))
    keys = keys_ref[...]

    km0 = _select_rank(keys, RANKS[0])
    ku0 = _select_rank(keys, RANKS[2])
    km0, km1 = _pair(keys, km0, RANKS[0])
    ku0, ku1 = _pair(keys, ku0, RANKS[2])

    new_med = 0.5 * (_key_to_f32(km0) + _key_to_f32(km1))
    new_upp = _key_to_f32(ku0) * (1.0 - FRAC_) + _key_to_f32(ku1) * FRAC_

    bpow = sc_ref[0]
    dive = sc_ref[1]
    trig = sc_ref[2]

    med = med_ref[...].reshape(1, BC_)
    upp = upp_ref[...].reshape(1, BC_)
    am, as_ = _ema_epilogue(new_med, new_upp, med, upp, bpow, dive, trig)
    om_ref[...] = am.reshape(BC_)
    os_ref[...] = as_.reshape(BC_)


def _tc_run(xt, med, upp, scalars):
    grid = H_TC // BC_
    return pl.pallas_call(
        _tc_body,
        grid=(grid,),
        in_specs=[
            pl.BlockSpec((N_, BC_), lambda i: (0, i)),
            pl.BlockSpec((BC_,), lambda i: (i,)),
            pl.BlockSpec((BC_,), lambda i: (i,)),
            pl.BlockSpec(memory_space=pltpu.SMEM),
        ],
        out_specs=[
            pl.BlockSpec((BC_,), lambda i: (i,)),
            pl.BlockSpec((BC_,), lambda i: (i,)),
        ],
        out_shape=[
            jax.ShapeDtypeStruct((H_TC,), jnp.float32),
            jax.ShapeDtypeStruct((H_TC,), jnp.float32),
        ],
        scratch_shapes=[pltpu.VMEM((N_, BC_), jnp.int32)],
    )(xt, med, upp, scalars)


@jax.jit
def _run(xr, med, upp, scal_sc, scal_tc):
    xs = xr[:, :H_SC]
    xt = xr[:, H_SC:]
    sm, ss = _sc_run(xs, med[:H_SC], upp[:H_SC], scal_sc)
    tm, ts = _tc_run(xt, med[H_SC:], upp[H_SC:], scal_tc)
    return (jnp.concatenate([sm, tm]), jnp.concatenate([ss, ts]))


def kernel(x, med, upp, steps, beta):
    xr = x[:4].reshape(N_, H_).astype(jnp.float32)
    delta = 1.0
    bpow = beta ** delta
    trig = (steps > 1.0).astype(jnp.float32)
    steps_f = jnp.where(steps > 1.0, steps, steps + delta)
    dive = 1.0 - beta ** steps_f + EPS_
    scal_sc = jnp.concatenate([
        jnp.broadcast_to(bpow, (L,)),
        jnp.broadcast_to(dive, (L,)),
        jnp.broadcast_to(trig, (L,)),
    ]).astype(jnp.float32)
    scal_tc = jnp.stack([bpow, dive, trig]).astype(jnp.float32)
    return _run(xr, med, upp, scal_sc, scal_tc)
